# baseline (speedup 1.0000x reference)
import functools

import jax
import jax.numpy as jnp
from jax.experimental import pallas as pl
from jax.experimental.pallas import tpu as pltpu

_K = 3277
_V = 32768
_ROWS = 32
_MININT = -2147483648


_CHUNK = 512
_C16 = 2048


def _lane_reduce(v):
    while v.shape[1] > 128:
        h = v.shape[1] // 2
        v = v[:, :h] + v[:, h:]
    return jnp.sum(v, axis=1, keepdims=True)


def _count_pred(s_ref, pred):
    r = s_ref.shape[0]
    acc = jnp.zeros((r, _CHUNK), jnp.int32)
    for c in range(_V // _CHUNK):
        sl = s_ref[:, c * _CHUNK : (c + 1) * _CHUNK]
        acc = acc + pred(sl, c).astype(jnp.int32)
    return _lane_reduce(acc)


def _count16(ref, pred):
    r = ref.shape[0]
    acc = jnp.zeros((r, _C16), jnp.int16)
    for c in range(_V // _C16):
        sl = ref[:, c * _C16 : (c + 1) * _C16]
        acc = acc + pred(sl).astype(jnp.int16)
    return _lane_reduce(acc.astype(jnp.int32))


def _greedy16(ref, target, n_ge_init):
    def step(i, carry):
        t_u, n_ge = carry
        bit = jnp.left_shift(jnp.int32(1), jnp.int32(15) - i)
        cand_u = t_u | bit
        cand16 = (cand_u - 32768).astype(jnp.int16)
        cnt = _count16(ref, lambda sl: sl >= cand16)
        acc = cnt >= target
        return jnp.where(acc, cand_u, t_u), jnp.where(acc, cnt, n_ge)

    r = ref.shape[0]
    return jax.lax.fori_loop(
        0, 16, step, (jnp.zeros((r, 1), jnp.int32), n_ge_init)
    )


def _topk_mask_body(x_ref, o_ref, s_ref, h_ref, l_ref):
    b = jax.lax.bitcast_convert_type(x_ref[...], jnp.int32)
    s = jnp.where(b < 0, b ^ jnp.int32(0x7FFFFFFF), b)
    s_ref[...] = s
    h_ref[...] = jax.lax.shift_right_arithmetic(s, 16).astype(jnp.int16)
    r = x_ref.shape[0]

    t_uh, n_ge_h = _greedy16(
        h_ref, jnp.int32(_K), jnp.full((r, 1), _V, jnp.int32)
    )
    p16 = (t_uh - 32768).astype(jnp.int16)
    a_gt = _count16(h_ref, lambda sl: sl > p16)
    k_lo = _K - a_gt

    s2 = s_ref[...]
    lo_b = ((s2 & jnp.int32(0xFFFF)) - 32768).astype(jnp.int16)
    h_eq = jax.lax.shift_right_arithmetic(s2, 16).astype(jnp.int16) == p16
    l_ref[...] = jnp.where(h_eq, lo_b, jnp.int16(-32768))

    t_ul, n_ge_l = _greedy16(l_ref, k_lo, n_ge_h - a_gt)
    n_ge = a_gt + n_ge_l
    t_s = jnp.left_shift(t_uh - 32768, 16) | t_ul

    def tie_path():
        cnt_eq = _count_pred(s_ref, lambda sl, c: sl == t_s)
        need = _K - (n_ge - cnt_eq)

        def idx_step(i, res):
            bit = jnp.left_shift(jnp.int32(1), jnp.int32(15) - i)
            cand = res | bit

            def pred(sl, c):
                idx = jax.lax.broadcasted_iota(
                    jnp.int32, sl.shape, 1
                ) + jnp.int32(c * _CHUNK)
                return (sl == t_s) & (idx < cand)

            g = _count_pred(s_ref, pred)
            return jnp.where(g <= need, cand, res)

        res = jax.lax.fori_loop(
            0, 16, idx_step, jnp.zeros((r, 1), jnp.int32)
        )
        s = s_ref[...]
        idx = jax.lax.broadcasted_iota(jnp.int32, s.shape, 1)
        keep = (s > t_s) | ((s == t_s) & (idx < res))
        return jnp.where(keep, x_ref[...], jnp.float32(-jnp.inf))

    def fast_path():
        return jnp.where(
            s_ref[...] >= t_s, x_ref[...], jnp.float32(-jnp.inf)
        )

    o_ref[...] = jax.lax.cond(jnp.any(n_ge != _K), tie_path, fast_path)


@functools.partial(jax.jit, static_argnums=())
def _topk_mask(flat):
    n_rows = flat.shape[0]
    return pl.pallas_call(
        _topk_mask_body,
        grid=(n_rows // _ROWS,),
        in_specs=[pl.BlockSpec((_ROWS, _V), lambda i: (i, 0))],
        out_specs=pl.BlockSpec((_ROWS, _V), lambda i: (i, 0)),
        out_shape=jax.ShapeDtypeStruct((n_rows, _V), jnp.float32),
        scratch_shapes=[
            pltpu.VMEM((_ROWS, _V), jnp.int32),
            pltpu.VMEM((_ROWS, _V), jnp.int16),
            pltpu.VMEM((_ROWS, _V), jnp.int16),
        ],
        compiler_params=pltpu.CompilerParams(
            dimension_semantics=("parallel",),
        ),
    )(flat)


def kernel(logits, k):
    B, S, V = logits.shape
    out = _topk_mask(logits.reshape(B * S, V))
    return out.reshape(B, S, V)


# speedup vs baseline: 1.0552x; 1.0552x over previous
import functools

import jax
import jax.numpy as jnp
from jax.experimental import pallas as pl
from jax.experimental.pallas import tpu as pltpu

_K = 3277
_V = 32768
_ROWS = 32
_MININT = -2147483648


_CHUNK = 512
_C16 = 1024


def _lane_reduce(v):
    while v.shape[1] > 128:
        h = v.shape[1] // 2
        v = v[:, :h] + v[:, h:]
    return jnp.sum(v, axis=1, keepdims=True)


def _count_pred(s_ref, pred):
    r = s_ref.shape[0]
    acc = jnp.zeros((r, _CHUNK), jnp.int32)
    for c in range(_V // _CHUNK):
        sl = s_ref[:, c * _CHUNK : (c + 1) * _CHUNK]
        acc = acc + pred(sl, c).astype(jnp.int32)
    return _lane_reduce(acc)


def _count16(ref, pred):
    r = ref.shape[0]
    acc = jnp.zeros((r, _C16), jnp.int16)
    for c in range(_V // _C16):
        sl = ref[:, c * _C16 : (c + 1) * _C16]
        acc = acc + pred(sl).astype(jnp.int16)
    return _lane_reduce(acc.astype(jnp.int32))


def _greedy16(ref, target, n_ge_init):
    def step(i, carry):
        t_u, n_ge = carry
        bit = jnp.left_shift(jnp.int32(1), jnp.int32(15) - i)
        cand_u = t_u | bit
        cand16 = (cand_u - 32768).astype(jnp.int16)
        cnt = _count16(ref, lambda sl: sl >= cand16)
        acc = cnt >= target
        return jnp.where(acc, cand_u, t_u), jnp.where(acc, cnt, n_ge)

    r = ref.shape[0]
    return jax.lax.fori_loop(
        0, 16, step, (jnp.zeros((r, 1), jnp.int32), n_ge_init)
    )


def _topk_mask_body(x_ref, o_ref, s_ref, h_ref, l_ref):
    b = jax.lax.bitcast_convert_type(x_ref[...], jnp.int32)
    s = jnp.where(b < 0, b ^ jnp.int32(0x7FFFFFFF), b)
    s_ref[...] = s
    h_ref[...] = jax.lax.shift_right_arithmetic(s, 16).astype(jnp.int16)
    r = x_ref.shape[0]

    t_uh, n_ge_h = _greedy16(
        h_ref, jnp.int32(_K), jnp.full((r, 1), _V, jnp.int32)
    )
    p16 = (t_uh - 32768).astype(jnp.int16)
    a_gt = _count16(h_ref, lambda sl: sl > p16)
    k_lo = _K - a_gt

    lo_b = ((s_ref[...] & jnp.int32(0xFFFF)) - 32768).astype(jnp.int16)
    l_ref[...] = jnp.where(h_ref[...] == p16, lo_b, jnp.int16(-32768))

    t_ul, n_ge_l = _greedy16(l_ref, k_lo, n_ge_h - a_gt)
    n_ge = a_gt + n_ge_l
    t_s = jnp.left_shift(t_uh - 32768, 16) | t_ul

    def tie_path():
        cnt_eq = _count_pred(s_ref, lambda sl, c: sl == t_s)
        need = _K - (n_ge - cnt_eq)

        def idx_step(i, res):
            bit = jnp.left_shift(jnp.int32(1), jnp.int32(15) - i)
            cand = res | bit

            def pred(sl, c):
                idx = jax.lax.broadcasted_iota(
                    jnp.int32, sl.shape, 1
                ) + jnp.int32(c * _CHUNK)
                return (sl == t_s) & (idx < cand)

            g = _count_pred(s_ref, pred)
            return jnp.where(g <= need, cand, res)

        res = jax.lax.fori_loop(
            0, 16, idx_step, jnp.zeros((r, 1), jnp.int32)
        )
        s = s_ref[...]
        idx = jax.lax.broadcasted_iota(jnp.int32, s.shape, 1)
        keep = (s > t_s) | ((s == t_s) & (idx < res))
        return jnp.where(keep, x_ref[...], jnp.float32(-jnp.inf))

    def fast_path():
        return jnp.where(
            s_ref[...] >= t_s, x_ref[...], jnp.float32(-jnp.inf)
        )

    o_ref[...] = jax.lax.cond(jnp.any(n_ge != _K), tie_path, fast_path)


@functools.partial(jax.jit, static_argnums=())
def _topk_mask(flat):
    n_rows = flat.shape[0]
    return pl.pallas_call(
        _topk_mask_body,
        grid=(n_rows // _ROWS,),
        in_specs=[pl.BlockSpec((_ROWS, _V), lambda i: (i, 0))],
        out_specs=pl.BlockSpec((_ROWS, _V), lambda i: (i, 0)),
        out_shape=jax.ShapeDtypeStruct((n_rows, _V), jnp.float32),
        scratch_shapes=[
            pltpu.VMEM((_ROWS, _V), jnp.int32),
            pltpu.VMEM((_ROWS, _V), jnp.int16),
            pltpu.VMEM((_ROWS, _V), jnp.int16),
        ],
        compiler_params=pltpu.CompilerParams(
            dimension_semantics=("parallel",),
        ),
    )(flat)


def kernel(logits, k):
    B, S, V = logits.shape
    out = _topk_mask(logits.reshape(B * S, V))
    return out.reshape(B, S, V)


# speedup vs baseline: 1.0555x; 1.0003x over previous
import functools

import jax
import jax.numpy as jnp
from jax.experimental import pallas as pl
from jax.experimental.pallas import tpu as pltpu

_K = 3277
_V = 32768
_ROWS = 32
_CHUNK = 512
_C16 = 1024


def _lane_reduce(v):
    while v.shape[1] > 128:
        h = v.shape[1] // 2
        v = v[:, :h] + v[:, h:]
    return jnp.sum(v, axis=1, keepdims=True)


def _count_pred(s_ref, pred):
    r = s_ref.shape[0]
    acc = jnp.zeros((r, _CHUNK), jnp.int32)
    for c in range(_V // _CHUNK):
        sl = s_ref[:, c * _CHUNK : (c + 1) * _CHUNK]
        acc = acc + pred(sl, c).astype(jnp.int32)
    return _lane_reduce(acc)


def _count16(ref, pred):
    r = ref.shape[0]
    acc = jnp.zeros((r, _C16), jnp.int16)
    for c in range(_V // _C16):
        sl = ref[:, c * _C16 : (c + 1) * _C16]
        acc = acc + pred(sl).astype(jnp.int16)
    return _lane_reduce(acc.astype(jnp.int32))


def _greedy16(ref, target, n_ge_init):
    def step(i, carry):
        t_u, n_ge = carry
        bit = jnp.left_shift(jnp.int32(1), jnp.int32(15) - i)
        cand_u = t_u | bit
        cand16 = (cand_u - 32768).astype(jnp.int16)
        cnt = _count16(ref, lambda sl: sl >= cand16)
        acc = cnt >= target
        return jnp.where(acc, cand_u, t_u), jnp.where(acc, cnt, n_ge)

    r = ref.shape[0]
    return jax.lax.fori_loop(
        0, 16, step, (jnp.zeros((r, 1), jnp.int32), n_ge_init)
    )


def _topk_mask_body(x_ref, o_ref, s_ref, h_ref, l_ref):
    b = jax.lax.bitcast_convert_type(x_ref[...], jnp.int32)
    s = jnp.where(b < 0, b ^ jnp.int32(0x7FFFFFFF), b)
    s_ref[...] = s
    h_ref[...] = jax.lax.shift_right_arithmetic(s, 16).astype(jnp.int16)
    r = x_ref.shape[0]

    t_uh, n_ge_h = _greedy16(
        h_ref, jnp.int32(_K), jnp.full((r, 1), _V, jnp.int32)
    )
    p16 = (t_uh - 32768).astype(jnp.int16)
    a_gt = _count16(h_ref, lambda sl: sl > p16)
    k_lo = _K - a_gt

    lo_b = ((s_ref[...] & jnp.int32(0xFFFF)) - 32768).astype(jnp.int16)
    l_ref[...] = jnp.where(h_ref[...] == p16, lo_b, jnp.int16(-32768))

    t_ul, n_ge_l = _greedy16(l_ref, k_lo, n_ge_h - a_gt)
    n_ge = a_gt + n_ge_l
    t_s = jnp.left_shift(t_uh - 32768, 16) | t_ul

    def tie_path():
        cnt_eq = _count_pred(s_ref, lambda sl, c: sl == t_s)
        need = _K - (n_ge - cnt_eq)

        def idx_step(i, res):
            bit = jnp.left_shift(jnp.int32(1), jnp.int32(15) - i)
            cand = res | bit

            def pred(sl, c):
                idx = jax.lax.broadcasted_iota(
                    jnp.int32, sl.shape, 1
                ) + jnp.int32(c * _CHUNK)
                return (sl == t_s) & (idx < cand)

            g = _count_pred(s_ref, pred)
            return jnp.where(g <= need, cand, res)

        res = jax.lax.fori_loop(
            0, 16, idx_step, jnp.zeros((r, 1), jnp.int32)
        )
        s = s_ref[...]
        idx = jax.lax.broadcasted_iota(jnp.int32, s.shape, 1)
        keep = (s > t_s) | ((s == t_s) & (idx < res))
        return jnp.where(keep, x_ref[...], jnp.float32(-jnp.inf))

    def fast_path():
        return jnp.where(
            s_ref[...] >= t_s, x_ref[...], jnp.float32(-jnp.inf)
        )

    o_ref[...] = jax.lax.cond(jnp.any(n_ge != _K), tie_path, fast_path)


@functools.partial(jax.jit, static_argnums=())
def _topk_mask(flat):
    n_rows = flat.shape[0]
    return pl.pallas_call(
        _topk_mask_body,
        grid=(n_rows // _ROWS,),
        in_specs=[pl.BlockSpec((_ROWS, _V), lambda i: (i, 0))],
        out_specs=pl.BlockSpec((_ROWS, _V), lambda i: (i, 0)),
        out_shape=jax.ShapeDtypeStruct((n_rows, _V), jnp.float32),
        scratch_shapes=[
            pltpu.VMEM((_ROWS, _V), jnp.int32),
            pltpu.VMEM((_ROWS, _V), jnp.int16),
            pltpu.VMEM((_ROWS, _V), jnp.int16),
        ],
        compiler_params=pltpu.CompilerParams(
            dimension_semantics=("parallel",),
        ),
    )(flat)


def kernel(logits, k):
    B, S, V = logits.shape
    out = _topk_mask(logits.reshape(B * S, V))
    return out.reshape(B, S, V)
